# baseline (device time: 66339 ns/iter reference)
import jax
import jax.numpy as jnp
from jax import lax
from jax.experimental import pallas as pl
from jax.experimental.pallas import tpu as pltpu

N_DEV = 4


def kernel(x, Win0, Wout0, Win1, Wout1, Win2, Wout2):
    m_per, d = x.shape

    def body(x_ref, win0_ref, wout0_ref, win1_ref, wout1_ref, win2_ref,
             wout2_ref, out_ref,
             xcur, ag1L, ag1R, ag2, rsA, rb_fL, rb_fR,
             pj, pL, pR, pD, stage, winb, woutb, ssem, rsem):
        j = lax.axis_index("i")
        left = lax.rem(j + N_DEV - 1, N_DEV)
        right = lax.rem(j + 1, N_DEV)

        barrier_sem = pltpu.get_barrier_semaphore()
        for nbr in (left, right):
            pl.semaphore_signal(barrier_sem, inc=1, device_id=(nbr,),
                                device_id_type=pl.DeviceIdType.MESH)
        pl.semaphore_wait(barrier_sem, 2)

        def mlp(src_ref, winb_ref, woutb_ref):
            h = jnp.maximum(
                jnp.dot(src_ref[...].astype(jnp.bfloat16), winb_ref[...],
                        preferred_element_type=jnp.float32), 0.0)
            return jnp.dot(h.astype(jnp.bfloat16), woutb_ref[...],
                           preferred_element_type=jnp.float32)

        def copy(src, dst, s, r, dev):
            return pltpu.make_async_remote_copy(
                src_ref=src, dst_ref=dst, send_sem=ssem.at[s],
                recv_sem=rsem.at[r], device_id=(dev,),
                device_id_type=pl.DeviceIdType.MESH)

        xcur[...] = x_ref[...]

        layers = [(win0_ref, wout0_ref), (win1_ref, wout1_ref),
                  (win2_ref, wout2_ref)]
        for l, (win_ref, wout_ref) in enumerate(layers):
            r1r = copy(xcur, ag1L, 0, 0, right)
            r1l = copy(xcur, ag1R, 1, 1, left)
            r1r.start()
            r1l.start()
            winb[...] = win_ref[...].astype(jnp.bfloat16)
            woutb[...] = wout_ref[...].astype(jnp.bfloat16)
            pj[...] = mlp(xcur, winb, woutb)
            r1r.wait()
            r1l.wait()

            r2 = copy(ag1L, ag2, 2, 2, right)
            r2.start()
            pL[...] = mlp(ag1L, winb, woutb)
            pR[...] = mlp(ag1R, winb, woutb)
            r2.wait()
            pD[...] = mlp(ag2, winb, woutb)

            ra = copy(pD, rsA, 3, 3, left)
            ra.start()
            ra.wait()

            stage[...] = pL[...] + rsA[...]
            rbl = copy(stage, rb_fR, 4, 4, left)
            rbr = copy(pR, rb_fL, 5, 5, right)
            rbl.start()
            rbr.start()
            rbl.wait()
            rbr.wait()

            res = pj[...] + rb_fL[...] + rb_fR[...]
            if l < len(layers) - 1:
                xcur[...] = res
            else:
                out_ref[...] = res

    buf = lambda: pltpu.VMEM((m_per, d), jnp.float32)
    return pl.pallas_call(
        body,
        out_shape=jax.ShapeDtypeStruct((m_per, d), jnp.float32),
        in_specs=[pl.BlockSpec(memory_space=pltpu.VMEM)] * 7,
        out_specs=pl.BlockSpec(memory_space=pltpu.VMEM),
        scratch_shapes=[
            buf(),
            buf(),
            buf(),
            buf(),
            buf(),
            buf(),
            buf(),
            buf(),
            buf(),
            buf(),
            buf(),
            buf(),
            pltpu.VMEM(Win0.shape, jnp.bfloat16),
            pltpu.VMEM(Wout0.shape, jnp.bfloat16),
            pltpu.SemaphoreType.DMA((6,)),
            pltpu.SemaphoreType.DMA((6,)),
        ],
        compiler_params=pltpu.CompilerParams(collective_id=0),
    )(x, Win0, Wout0, Win1, Wout1, Win2, Wout2)


# device time: 38997 ns/iter; 1.7011x vs baseline; 1.7011x over previous
import jax
import jax.numpy as jnp
from jax import lax
from jax.experimental import pallas as pl
from jax.experimental.pallas import tpu as pltpu

N_DEV = 4


def kernel(x, Win0, Wout0, Win1, Wout1, Win2, Wout2):
    m_per, d = x.shape

    def body(x_ref, win0_ref, wout0_ref, win1_ref, wout1_ref, win2_ref,
             wout2_ref, out_ref,
             xcur, xb, agL, agR, agD, pj, pLb, pRb, pDb,
             rsFromL, rsFromR, rsFromD, winb, woutb, ssem, rsem):
        j = lax.axis_index("i")
        left = lax.rem(j + N_DEV - 1, N_DEV)
        right = lax.rem(j + 1, N_DEV)
        diag = lax.rem(j + 2, N_DEV)

        barrier_sem = pltpu.get_barrier_semaphore()
        for nbr in (left, right, diag):
            pl.semaphore_signal(barrier_sem, inc=1, device_id=(nbr,),
                                device_id_type=pl.DeviceIdType.MESH)
        pl.semaphore_wait(barrier_sem, 3)

        def mlp(src_ref):
            h = jnp.maximum(
                jnp.dot(src_ref[...], winb[...],
                        preferred_element_type=jnp.float32), 0.0)
            return jnp.dot(h.astype(jnp.bfloat16), woutb[...],
                           preferred_element_type=jnp.float32)

        def copy(src, dst, s, r, dev):
            return pltpu.make_async_remote_copy(
                src_ref=src, dst_ref=dst, send_sem=ssem.at[s],
                recv_sem=rsem.at[r], device_id=(dev,),
                device_id_type=pl.DeviceIdType.MESH)

        xcur[...] = x_ref[...]

        layers = [(win0_ref, wout0_ref), (win1_ref, wout1_ref),
                  (win2_ref, wout2_ref)]
        for l, (win_ref, wout_ref) in enumerate(layers):
            xb[...] = xcur[...].astype(jnp.bfloat16)
            agl = copy(xb, agL, 0, 0, right)
            agr = copy(xb, agR, 1, 1, left)
            agd = copy(xb, agD, 2, 2, diag)
            agl.start()
            agr.start()
            agd.start()
            winb[...] = win_ref[...].astype(jnp.bfloat16)
            woutb[...] = wout_ref[...].astype(jnp.bfloat16)
            pj[...] = mlp(xb)

            agl.wait()
            pLb[...] = mlp(agL).astype(jnp.bfloat16)
            rsl = copy(pLb, rsFromR, 3, 3, left)
            rsl.start()
            agr.wait()
            pRb[...] = mlp(agR).astype(jnp.bfloat16)
            rsr = copy(pRb, rsFromL, 4, 4, right)
            rsr.start()
            agd.wait()
            pDb[...] = mlp(agD).astype(jnp.bfloat16)
            rsd = copy(pDb, rsFromD, 5, 5, diag)
            rsd.start()
            rsl.wait()
            rsr.wait()
            rsd.wait()

            res = (pj[...] + rsFromL[...].astype(jnp.float32)
                   + rsFromR[...].astype(jnp.float32)
                   + rsFromD[...].astype(jnp.float32))
            if l < len(layers) - 1:
                xcur[...] = res
            else:
                out_ref[...] = res

    bufb = lambda: pltpu.VMEM((m_per, d), jnp.bfloat16)
    return pl.pallas_call(
        body,
        out_shape=jax.ShapeDtypeStruct((m_per, d), jnp.float32),
        in_specs=[pl.BlockSpec(memory_space=pltpu.VMEM)] * 7,
        out_specs=pl.BlockSpec(memory_space=pltpu.VMEM),
        scratch_shapes=[
            pltpu.VMEM((m_per, d), jnp.float32),
            bufb(),
            bufb(),
            bufb(),
            bufb(),
            pltpu.VMEM((m_per, d), jnp.float32),
            bufb(),
            bufb(),
            bufb(),
            bufb(),
            bufb(),
            bufb(),
            pltpu.VMEM(Win0.shape, jnp.bfloat16),
            pltpu.VMEM(Wout0.shape, jnp.bfloat16),
            pltpu.SemaphoreType.DMA((6,)),
            pltpu.SemaphoreType.DMA((6,)),
        ],
        compiler_params=pltpu.CompilerParams(collective_id=0),
    )(x, Win0, Wout0, Win1, Wout1, Win2, Wout2)
